# baseline (speedup 1.0000x reference)
import functools

import jax
import jax.numpy as jnp
from jax.experimental import pallas as pl

B, V, D, P = 64, 7, 1024, 64
IN = D * P
OUT = 96
E, K, R = 16, 2, 8
H = D // 2
SCALING = 16 / R

N = B * V
PG = 8
NSTEPS = P // PG

_NT = (((1,), (1,)), ((), ()))


def _stage1_body(x_ref, wb_ref, a_ref, base_ref, temp_ref, pool_ref):
    i = pl.program_id(0)
    b_part = jnp.zeros((N, OUT), jnp.float32)
    t_part = jnp.zeros((N, E * R), jnp.float32)
    ps = jnp.zeros((B, D), jnp.float32)
    for q in range(PG):
        xq = x_ref[:, :, q, :]
        ps = ps + xq.sum(axis=1)
        xq16 = xq.reshape(N, D).astype(jnp.bfloat16)
        b_part += jax.lax.dot_general(
            xq16, wb_ref[:, q * D:(q + 1) * D], _NT,
            preferred_element_type=jnp.float32)
        t_part += jax.lax.dot_general(
            xq16, a_ref[:, q * D:(q + 1) * D], _NT,
            preferred_element_type=jnp.float32)

    @pl.when(i == 0)
    def _init():
        base_ref[...] = b_part
        temp_ref[...] = t_part
        pool_ref[...] = ps * (1.0 / (V * P))

    @pl.when(i != 0)
    def _acc():
        base_ref[...] += b_part
        temp_ref[...] += t_part
        pool_ref[...] += ps * (1.0 / (V * P))


def _stage2_body(base_ref, temp_ref, pool_ref, w1_ref, b1_ref, w2_ref,
                 b2_ref, bb_ref, lb_ref, out_ref, probs_ref):
    pooled = pool_ref[...]
    h = jax.lax.dot_general(pooled, w1_ref[...], _NT,
                            preferred_element_type=jnp.float32) + b1_ref[...]
    h = jnp.maximum(h, 0.0)
    logits = jax.lax.dot_general(h, w2_ref[...], _NT,
                                 preferred_element_type=jnp.float32) + b2_ref[...]
    m = jnp.max(logits, axis=-1, keepdims=True)
    ex = jnp.exp(logits - m)
    probs = ex / jnp.sum(ex, axis=-1, keepdims=True)
    probs_ref[...] = probs

    eidx = jax.lax.broadcasted_iota(jnp.int32, (B, E), 1)
    m1 = jnp.max(probs, axis=-1, keepdims=True)
    i1 = jnp.min(jnp.where(probs == m1, eidx, E), axis=-1, keepdims=True)
    masked = jnp.where(eidx == i1, -1.0, probs)
    m2 = jnp.max(masked, axis=-1, keepdims=True)
    i2 = jnp.min(jnp.where(masked == m2, eidx, E), axis=-1, keepdims=True)
    s = jnp.maximum(m1 + m2, 1e-6)
    w_e = (m1 / s) * (eidx == i1) + (m2 / s) * (eidx == i2)

    rn = jax.lax.broadcasted_iota(jnp.int32, (N, B), 0) // V
    cb = jax.lax.broadcasted_iota(jnp.int32, (N, B), 1)
    sel = (rn == cb).astype(jnp.float32)
    w_rows = jnp.dot(sel, w_e, preferred_element_type=jnp.float32)

    temp = temp_ref[...]
    moe = jnp.zeros((N, OUT), dtype=jnp.float32)
    for e in range(E):
        te = temp[:, e * R:(e + 1) * R]
        de = jax.lax.dot_general(te, lb_ref[e], _NT,
                                 preferred_element_type=jnp.float32)
        moe += w_rows[:, e:e + 1] * de
    out_ref[...] = base_ref[...] + bb_ref[...] + moe * SCALING


@functools.partial(jax.jit, static_argnames=("interpret",))
def _run(x, W_base, b_base, W1, b1, W2, b2, lora_A, lora_B, interpret=False):
    xt = jnp.transpose(x, (0, 1, 3, 2))
    wt = jnp.transpose(W_base.reshape(OUT, D, P), (0, 2, 1))
    wt = wt.reshape(OUT, IN).astype(jnp.bfloat16)
    at = jnp.transpose(lora_A.reshape(E * R, D, P), (0, 2, 1))
    at = at.reshape(E * R, IN).astype(jnp.bfloat16)

    base_acc, temp_acc, pooled = pl.pallas_call(
        _stage1_body,
        grid=(NSTEPS,),
        in_specs=[
            pl.BlockSpec((B, V, PG, D), lambda i: (0, 0, i, 0)),
            pl.BlockSpec((OUT, PG * D), lambda i: (0, i)),
            pl.BlockSpec((E * R, PG * D), lambda i: (0, i)),
        ],
        out_specs=[
            pl.BlockSpec((N, OUT), lambda i: (0, 0)),
            pl.BlockSpec((N, E * R), lambda i: (0, 0)),
            pl.BlockSpec((B, D), lambda i: (0, 0)),
        ],
        out_shape=[
            jax.ShapeDtypeStruct((N, OUT), jnp.float32),
            jax.ShapeDtypeStruct((N, E * R), jnp.float32),
            jax.ShapeDtypeStruct((B, D), jnp.float32),
        ],
        interpret=interpret,
    )(xt, wt, at)

    final, probs = pl.pallas_call(
        _stage2_body,
        out_shape=[
            jax.ShapeDtypeStruct((N, OUT), jnp.float32),
            jax.ShapeDtypeStruct((B, E), jnp.float32),
        ],
        interpret=interpret,
    )(base_acc, temp_acc, pooled, W1, b1.reshape(1, H), W2,
      b2.reshape(1, E), b_base.reshape(1, OUT), lora_B)
    return final.reshape(B, V, OUT), probs


def kernel(x, W_base, b_base, W1, b1, W2, b2, lora_A, lora_B):
    return _run(x, W_base, b_base, W1, b1, W2, b2, lora_A, lora_B)


# speedup vs baseline: 3.9650x; 3.9650x over previous
import functools

import jax
import jax.numpy as jnp
from jax.experimental import pallas as pl

B, V, D, P = 64, 7, 1024, 64
IN = D * P
OUT = 96
E, K, R = 16, 2, 8
H = D // 2
SCALING = 16 / R

N = B * V
DC = 128
NSTEPS = D // DC
CHUNK = DC * P

_NT = (((1,), (1,)), ((), ()))


def _stage1_body(x_ref, wb_ref, a_ref, base_ref, temp_ref, pool_ref):
    i = pl.program_id(0)
    x4 = x_ref[...]
    x16 = x4.astype(jnp.bfloat16).swapaxes(2, 3)
    xb16 = x16.reshape(N, CHUNK)
    b_part = jax.lax.dot_general(xb16, wb_ref[...].astype(jnp.bfloat16), _NT,
                                 preferred_element_type=jnp.float32)
    t_part = jax.lax.dot_general(xb16, a_ref[...].astype(jnp.bfloat16), _NT,
                                 preferred_element_type=jnp.float32)
    ps = x4.sum(axis=2).sum(axis=1)
    pool_ref[0] = ps * (1.0 / (V * P))

    @pl.when(i == 0)
    def _init():
        base_ref[...] = b_part
        temp_ref[...] = t_part

    @pl.when(i != 0)
    def _acc():
        base_ref[...] += b_part
        temp_ref[...] += t_part


def _stage2_body(base_ref, temp_ref, pool_ref, w1_ref, b1_ref, w2_ref,
                 b2_ref, bb_ref, lb_ref, out_ref, probs_ref):
    pooled = pool_ref[...]
    h = jax.lax.dot_general(pooled, w1_ref[...], _NT,
                            preferred_element_type=jnp.float32) + b1_ref[...]
    h = jnp.maximum(h, 0.0)
    logits = jax.lax.dot_general(h, w2_ref[...], _NT,
                                 preferred_element_type=jnp.float32) + b2_ref[...]
    m = jnp.max(logits, axis=-1, keepdims=True)
    ex = jnp.exp(logits - m)
    probs = ex / jnp.sum(ex, axis=-1, keepdims=True)
    probs_ref[...] = probs

    eidx = jax.lax.broadcasted_iota(jnp.int32, (B, E), 1)
    m1 = jnp.max(probs, axis=-1, keepdims=True)
    i1 = jnp.min(jnp.where(probs == m1, eidx, E), axis=-1, keepdims=True)
    masked = jnp.where(eidx == i1, -1.0, probs)
    m2 = jnp.max(masked, axis=-1, keepdims=True)
    i2 = jnp.min(jnp.where(masked == m2, eidx, E), axis=-1, keepdims=True)
    s = jnp.maximum(m1 + m2, 1e-6)
    w_e = (m1 / s) * (eidx == i1) + (m2 / s) * (eidx == i2)

    rn = jax.lax.broadcasted_iota(jnp.int32, (N, B), 0) // V
    cb = jax.lax.broadcasted_iota(jnp.int32, (N, B), 1)
    sel = (rn == cb).astype(jnp.float32)
    w_rows = jnp.dot(sel, w_e, preferred_element_type=jnp.float32)

    temp = temp_ref[...]
    moe = jnp.zeros((N, OUT), dtype=jnp.float32)
    for e in range(E):
        te = temp[:, e * R:(e + 1) * R]
        de = jax.lax.dot_general(te, lb_ref[e], _NT,
                                 preferred_element_type=jnp.float32)
        moe += w_rows[:, e:e + 1] * de
    out_ref[...] = base_ref[...] + bb_ref[...] + moe * SCALING


@functools.partial(jax.jit, static_argnames=("interpret",))
def _run(x, W_base, b_base, W1, b1, W2, b2, lora_A, lora_B, interpret=False):
    xt = jnp.transpose(x, (0, 1, 3, 2))
    A2 = lora_A.reshape(E * R, IN)

    base_acc, temp_acc, pooled = pl.pallas_call(
        _stage1_body,
        grid=(NSTEPS,),
        in_specs=[
            pl.BlockSpec((B, V, P, DC), lambda i: (0, 0, 0, i)),
            pl.BlockSpec((OUT, CHUNK), lambda i: (0, i)),
            pl.BlockSpec((E * R, CHUNK), lambda i: (0, i)),
        ],
        out_specs=[
            pl.BlockSpec((N, OUT), lambda i: (0, 0)),
            pl.BlockSpec((N, E * R), lambda i: (0, 0)),
            pl.BlockSpec((1, B, DC), lambda i: (i, 0, 0)),
        ],
        out_shape=[
            jax.ShapeDtypeStruct((N, OUT), jnp.float32),
            jax.ShapeDtypeStruct((N, E * R), jnp.float32),
            jax.ShapeDtypeStruct((NSTEPS, B, DC), jnp.float32),
        ],
        interpret=interpret,
    )(xt, W_base, A2)
    pooled = pooled.transpose(1, 0, 2).reshape(B, D)

    final, probs = pl.pallas_call(
        _stage2_body,
        out_shape=[
            jax.ShapeDtypeStruct((N, OUT), jnp.float32),
            jax.ShapeDtypeStruct((B, E), jnp.float32),
        ],
        interpret=interpret,
    )(base_acc, temp_acc, pooled, W1, b1.reshape(1, H), W2,
      b2.reshape(1, E), b_base.reshape(1, OUT), lora_B)
    return final.reshape(B, V, OUT), probs


def kernel(x, W_base, b_base, W1, b1, W2, b2, lora_A, lora_B):
    return _run(x, W_base, b_base, W1, b1, W2, b2, lora_A, lora_B)
